# Initial kernel scaffold; baseline (speedup 1.0000x reference)
import jax
import jax.numpy as jnp
from jax.experimental import pallas as pl


def kernel(x, edge_index, W1, b1, W2, b2, Wf, bf):
    raise NotImplementedError("write your pallas kernel here")


# speedup vs baseline: 12.6102x; 12.6102x over previous
import functools

import jax
import jax.numpy as jnp
from jax import lax
from jax.experimental import pallas as pl
from jax.experimental.pallas import tpu as pltpu
from jax.experimental.pallas import tpu_sc as plsc

_NC = 2
_NS = 16
_K = 80


@functools.lru_cache(maxsize=None)
def _make_deg_kernel(npad: int, e: int, k: int):
    ept = e // (_NC * _NS)
    rpt = npad // _NS
    mesh = plsc.VectorSubcoreMesh(core_axis_name="c", subcore_axis_name="s")

    @functools.partial(
        pl.kernel,
        mesh=mesh,
        out_type=jax.ShapeDtypeStruct((_NC * npad,), jnp.float32),
        scratch_types=[
            pltpu.VMEM((k,), jnp.int32),
            pltpu.VMEM((k,), jnp.float32),
            pltpu.VMEM_SHARED((npad,), jnp.float32),
            pltpu.SemaphoreType.DMA,
        ],
    )
    def deg_k(dst_hbm, ones_hbm, zero_hbm, out_hbm, dstv, onesv, degs, sem):
        cid = lax.axis_index("c")
        sid = lax.axis_index("s")
        r0 = sid * rpt
        pltpu.sync_copy(zero_hbm, degs.at[pl.ds(r0, rpt)])
        pltpu.sync_copy(ones_hbm, onesv)
        plsc.subcore_barrier()
        ebase = cid * (e // _NC) + sid * ept

        def body(i, carry):
            off = pl.multiple_of(ebase + i * k, 8)
            pltpu.sync_copy(dst_hbm.at[pl.ds(off, k)], dstv)
            pltpu.sync_copy(onesv, degs.at[dstv], add=True)
            return carry

        lax.fori_loop(0, ept // k, body, 0)
        plsc.subcore_barrier()
        pltpu.sync_copy(
            degs.at[pl.ds(r0, rpt)],
            out_hbm.at[pl.ds(cid * npad + r0, rpt)],
        )

    return deg_k


@functools.lru_cache(maxsize=None)
def _make_edge_kernel(npad: int, e: int, h: int, k: int):
    ept = e // (_NC * _NS)
    rpt = npad // _NS
    mesh = plsc.VectorSubcoreMesh(core_axis_name="c", subcore_axis_name="s")

    @functools.partial(
        pl.kernel,
        mesh=mesh,
        out_type=jax.ShapeDtypeStruct((_NC * npad, h), jnp.float32),
        scratch_types=[
            pltpu.VMEM((k,), jnp.int32),
            pltpu.VMEM((k,), jnp.int32),
            pltpu.VMEM((k, h), jnp.float32),
            pltpu.VMEM_SHARED((npad, h), jnp.float32),
            pltpu.SemaphoreType.DMA,
        ],
    )
    def edge_k(g_hbm, src_hbm, dst_hbm, zero_hbm, out_hbm,
               srcv, dstv, rows, aggs, sem):
        cid = lax.axis_index("c")
        sid = lax.axis_index("s")
        r0 = sid * rpt
        pltpu.sync_copy(zero_hbm, aggs.at[pl.ds(r0, rpt)])
        plsc.subcore_barrier()
        ebase = cid * (e // _NC) + sid * ept

        def body(i, carry):
            off = pl.multiple_of(ebase + i * k, 8)
            pltpu.sync_copy(src_hbm.at[pl.ds(off, k)], srcv)
            pltpu.sync_copy(dst_hbm.at[pl.ds(off, k)], dstv)
            pltpu.async_copy(g_hbm.at[srcv], rows, sem).wait()
            pltpu.sync_copy(rows, aggs.at[dstv], add=True)
            return carry

        lax.fori_loop(0, ept // k, body, 0)
        plsc.subcore_barrier()
        pltpu.sync_copy(
            aggs.at[pl.ds(r0, rpt)],
            out_hbm.at[pl.ds(cid * npad + r0, rpt)],
        )

    return edge_k


def _tc1_body(d0_ref, d1_ref, x_ref, w_ref, dis_ref, g_ref):
    deg = d0_ref[...] + d1_ref[...] + 1.0
    dis = lax.rsqrt(deg)
    dis_ref[...] = dis
    g_ref[...] = (
        jnp.dot(x_ref[...], w_ref[...], preferred_element_type=jnp.float32)
        * dis
    )


def _tc2_body(p0_ref, p1_ref, g_ref, dis_ref, b_ref, w_ref, out_ref):
    agg = p0_ref[...] + p1_ref[...] + g_ref[...]
    z = jnp.maximum(agg * dis_ref[...] + b_ref[...], 0.0)
    out_ref[...] = (
        jnp.dot(z, w_ref[...], preferred_element_type=jnp.float32)
        * dis_ref[...]
    )


def _tc3_body(p0_ref, p1_ref, g_ref, dis_ref, b_ref, wf_ref, bf_ref, out_ref):
    agg = p0_ref[...] + p1_ref[...] + g_ref[...]
    z = jnp.maximum(agg * dis_ref[...] + b_ref[...], 0.0)
    logits = (
        jnp.dot(z, wf_ref[...], preferred_element_type=jnp.float32)
        + bf_ref[...]
    )
    m = jnp.max(logits, axis=1, keepdims=True)
    s = logits - m
    out_ref[...] = s - jnp.log(jnp.sum(jnp.exp(s), axis=1, keepdims=True))


def _row_block(n):
    for b in (1000, 500, 250, 200, 128, 100, 50, 40, 8):
        if n % b == 0 and b % 8 == 0:
            return b
    return n


def kernel(x, edge_index, W1, b1, W2, b2, Wf, bf):
    n, d = x.shape
    h = W1.shape[1]
    c = Wf.shape[1]
    e = edge_index.shape[1]
    src = edge_index[0]
    dst = edge_index[1]

    npad = -(-n // 2048) * 2048
    b = _row_block(n)
    grid = (n // b,)

    ones_k = jnp.ones((_K,), jnp.float32)
    zero1 = jnp.zeros((npad // _NS,), jnp.float32)
    zero2 = jnp.zeros((npad // _NS, h), jnp.float32)

    degf = _make_deg_kernel(npad, e, _K)(dst, ones_k, zero1)
    d0 = degf[:n].reshape(n, 1)
    d1 = degf[npad:npad + n].reshape(n, 1)

    dis, g1 = pl.pallas_call(
        _tc1_body,
        grid=grid,
        in_specs=[
            pl.BlockSpec((b, 1), lambda i: (i, 0)),
            pl.BlockSpec((b, 1), lambda i: (i, 0)),
            pl.BlockSpec((b, d), lambda i: (i, 0)),
            pl.BlockSpec((d, h), lambda i: (0, 0)),
        ],
        out_specs=[
            pl.BlockSpec((b, 1), lambda i: (i, 0)),
            pl.BlockSpec((b, h), lambda i: (i, 0)),
        ],
        out_shape=[
            jax.ShapeDtypeStruct((n, 1), jnp.float32),
            jax.ShapeDtypeStruct((n, h), jnp.float32),
        ],
    )(d0, d1, x, W1)

    edge_k = _make_edge_kernel(npad, e, h, _K)

    p = edge_k(g1, src, dst, zero2)

    g2 = pl.pallas_call(
        _tc2_body,
        grid=grid,
        in_specs=[
            pl.BlockSpec((b, h), lambda i: (i, 0)),
            pl.BlockSpec((b, h), lambda i: (i, 0)),
            pl.BlockSpec((b, h), lambda i: (i, 0)),
            pl.BlockSpec((b, 1), lambda i: (i, 0)),
            pl.BlockSpec((1, h), lambda i: (0, 0)),
            pl.BlockSpec((h, h), lambda i: (0, 0)),
        ],
        out_specs=pl.BlockSpec((b, h), lambda i: (i, 0)),
        out_shape=jax.ShapeDtypeStruct((n, h), jnp.float32),
    )(p[:n], p[npad:npad + n], g1, dis, b1.reshape(1, h), W2)

    p2 = edge_k(g2, src, dst, zero2)

    out = pl.pallas_call(
        _tc3_body,
        grid=grid,
        in_specs=[
            pl.BlockSpec((b, h), lambda i: (i, 0)),
            pl.BlockSpec((b, h), lambda i: (i, 0)),
            pl.BlockSpec((b, h), lambda i: (i, 0)),
            pl.BlockSpec((b, 1), lambda i: (i, 0)),
            pl.BlockSpec((1, h), lambda i: (0, 0)),
            pl.BlockSpec((d, c), lambda i: (0, 0)),
            pl.BlockSpec((1, c), lambda i: (0, 0)),
        ],
        out_specs=pl.BlockSpec((b, c), lambda i: (i, 0)),
        out_shape=jax.ShapeDtypeStruct((n, c), jnp.float32),
    )(p2[:n], p2[npad:npad + n], g2, dis, b2.reshape(1, h), Wf, bf.reshape(1, c))

    return out


# speedup vs baseline: 27.5024x; 2.1810x over previous
import functools

import jax
import jax.numpy as jnp
from jax import lax
from jax.experimental import pallas as pl
from jax.experimental.pallas import tpu as pltpu
from jax.experimental.pallas import tpu_sc as plsc

_NC = 2
_NS = 16
_K = 80


@functools.lru_cache(maxsize=None)
def _make_deg_kernel(npad: int, e: int, k: int):
    ept = e // (_NC * _NS)
    nbatch = ept // k
    rpt = npad // _NS
    mesh = plsc.VectorSubcoreMesh(core_axis_name="c", subcore_axis_name="s")

    @functools.partial(
        pl.kernel,
        mesh=mesh,
        out_type=jax.ShapeDtypeStruct((_NC * npad,), jnp.float32),
        scratch_types=[
            pltpu.VMEM((_NB, k), jnp.int32),
            pltpu.VMEM((k,), jnp.float32),
            pltpu.VMEM_SHARED((npad,), jnp.float32),
        ] + [pltpu.SemaphoreType.DMA] * _NB,
    )
    def deg_k(dst_hbm, ones_hbm, zero_hbm, out_hbm, dstv, onesv, degs, *isem):
        cid = lax.axis_index("c")
        sid = lax.axis_index("s")
        r0 = sid * rpt
        pltpu.sync_copy(zero_hbm, degs.at[pl.ds(r0, rpt)])
        pltpu.sync_copy(ones_hbm, onesv)
        plsc.subcore_barrier()
        ebase = cid * (e // _NC) + sid * ept

        def start_idx(b, slot):
            off = pl.multiple_of(ebase + b * k, 8)
            pltpu.async_copy(dst_hbm.at[pl.ds(off, k)], dstv.at[slot],
                             isem[slot])

        def wait_idx(slot):
            pltpu.make_async_copy(dst_hbm.at[pl.ds(ebase, k)],
                                  dstv.at[slot], isem[slot]).wait()

        for b in range(_NB - 1):
            start_idx(b, b)

        def round_body(i, carry):
            b0 = i * _NB
            for s in range(_NB):
                start_idx(b0 + s + _NB - 1, (s + _NB - 1) % _NB)
                wait_idx(s)
                pltpu.sync_copy(onesv, degs.at[dstv.at[s]], add=True)
            return carry

        nround = (nbatch - (_NB - 1)) // _NB
        lax.fori_loop(0, nround, round_body, 0)
        for b in range(nround * _NB, nbatch):
            s = b % _NB
            if b + _NB - 1 < nbatch:
                start_idx(b + _NB - 1, (s + _NB - 1) % _NB)
            wait_idx(s)
            pltpu.sync_copy(onesv, degs.at[dstv.at[s]], add=True)

        plsc.subcore_barrier()
        pltpu.sync_copy(
            degs.at[pl.ds(r0, rpt)],
            out_hbm.at[pl.ds(cid * npad + r0, rpt)],
        )

    return deg_k


_NB = 4


@functools.lru_cache(maxsize=None)
def _make_edge_kernel(npad: int, e: int, h: int, k: int):
    ept = e // (_NC * _NS)
    rpt = npad // _NS
    nbatch = ept // k
    nround = nbatch // _NB - 1
    mesh = plsc.VectorSubcoreMesh(core_axis_name="c", subcore_axis_name="s")

    @functools.partial(
        pl.kernel,
        mesh=mesh,
        out_type=jax.ShapeDtypeStruct((_NC * npad, h), jnp.float32),
        scratch_types=[
            pltpu.VMEM((_NB, k), jnp.int32),
            pltpu.VMEM((_NB, k), jnp.int32),
            pltpu.VMEM((_NB, k, h), jnp.float32),
            pltpu.VMEM_SHARED((npad, h), jnp.float32),
        ] + [pltpu.SemaphoreType.DMA] * (2 * _NB),
    )
    def edge_k(g_hbm, src_hbm, dst_hbm, zero_hbm, out_hbm,
               srcv, dstv, rows, aggs, *sems):
        gsem = sems[:_NB]
        isem = sems[_NB:]
        cid = lax.axis_index("c")
        sid = lax.axis_index("s")
        r0 = sid * rpt
        pltpu.sync_copy(zero_hbm, aggs.at[pl.ds(r0, rpt)])
        plsc.subcore_barrier()
        ebase = cid * (e // _NC) + sid * ept

        def start_idx(b, slot):
            off = pl.multiple_of(ebase + b * k, 8)
            pltpu.async_copy(src_hbm.at[pl.ds(off, k)], srcv.at[slot],
                             isem[slot])
            pltpu.async_copy(dst_hbm.at[pl.ds(off, k)], dstv.at[slot],
                             isem[slot])

        def wait_idx(slot):
            pltpu.make_async_copy(src_hbm.at[pl.ds(ebase, k)],
                                  srcv.at[slot], isem[slot]).wait()
            pltpu.make_async_copy(dst_hbm.at[pl.ds(ebase, k)],
                                  dstv.at[slot], isem[slot]).wait()

        def start_gather(slot):
            pltpu.async_copy(g_hbm.at[srcv.at[slot]], rows.at[slot],
                             gsem[slot])

        def wait_gather(slot):
            pltpu.make_async_copy(g_hbm.at[srcv.at[slot]], rows.at[slot],
                                  gsem[slot]).wait()

        def scatter(slot):
            pltpu.sync_copy(rows.at[slot], aggs.at[dstv.at[slot]],
                            add=True)

        start_idx(0, 0)
        start_idx(1, 1)
        start_idx(2, 2)
        wait_idx(0)
        start_gather(0)

        def step(b, s):
            wait_idx((s + 1) % _NB)
            start_gather((s + 1) % _NB)
            wait_gather(s)
            scatter(s)

        def round_body(i, carry):
            b0 = i * _NB
            for s in range(_NB):
                start_idx(b0 + s + 3, (s + 3) % _NB)
                step(b0 + s, s)
            return carry

        lax.fori_loop(0, nround, round_body, 0)

        for b in range(nround * _NB, nbatch):
            s = b % _NB
            if b + 3 < nbatch:
                start_idx(b + 3, (s + 3) % _NB)
            if b + 1 < nbatch:
                wait_idx((s + 1) % _NB)
                start_gather((s + 1) % _NB)
            wait_gather(s)
            scatter(s)

        plsc.subcore_barrier()
        pltpu.sync_copy(
            aggs.at[pl.ds(r0, rpt)],
            out_hbm.at[pl.ds(cid * npad + r0, rpt)],
        )

    return edge_k


def _tc1_body(d0_ref, d1_ref, x_ref, w_ref, dis_ref, g_ref):
    deg = d0_ref[...] + d1_ref[...] + 1.0
    dis = lax.rsqrt(deg)
    dis_ref[...] = dis
    g_ref[...] = (
        jnp.dot(x_ref[...], w_ref[...], preferred_element_type=jnp.float32)
        * dis
    )


def _tc2_body(p0_ref, p1_ref, g_ref, dis_ref, b_ref, w_ref, out_ref):
    agg = p0_ref[...] + p1_ref[...] + g_ref[...]
    z = jnp.maximum(agg * dis_ref[...] + b_ref[...], 0.0)
    out_ref[...] = (
        jnp.dot(z, w_ref[...], preferred_element_type=jnp.float32)
        * dis_ref[...]
    )


def _tc3_body(p0_ref, p1_ref, g_ref, dis_ref, b_ref, wf_ref, bf_ref, out_ref):
    agg = p0_ref[...] + p1_ref[...] + g_ref[...]
    z = jnp.maximum(agg * dis_ref[...] + b_ref[...], 0.0)
    logits = (
        jnp.dot(z, wf_ref[...], preferred_element_type=jnp.float32)
        + bf_ref[...]
    )
    m = jnp.max(logits, axis=1, keepdims=True)
    s = logits - m
    out_ref[...] = s - jnp.log(jnp.sum(jnp.exp(s), axis=1, keepdims=True))


def _row_block(n):
    for b in (1000, 500, 250, 200, 128, 100, 50, 40, 8):
        if n % b == 0 and b % 8 == 0:
            return b
    return n


def kernel(x, edge_index, W1, b1, W2, b2, Wf, bf):
    n, d = x.shape
    h = W1.shape[1]
    c = Wf.shape[1]
    e = edge_index.shape[1]
    src = edge_index[0]
    dst = edge_index[1]

    npad = -(-n // 2048) * 2048
    b = _row_block(n)
    grid = (n // b,)

    ones_k = jnp.ones((_K,), jnp.float32)
    zero1 = jnp.zeros((npad // _NS,), jnp.float32)
    zero2 = jnp.zeros((npad // _NS, h), jnp.float32)

    degf = _make_deg_kernel(npad, e, _K)(dst, ones_k, zero1)
    d0 = degf[:n].reshape(n, 1)
    d1 = degf[npad:npad + n].reshape(n, 1)

    dis, g1 = pl.pallas_call(
        _tc1_body,
        grid=grid,
        in_specs=[
            pl.BlockSpec((b, 1), lambda i: (i, 0)),
            pl.BlockSpec((b, 1), lambda i: (i, 0)),
            pl.BlockSpec((b, d), lambda i: (i, 0)),
            pl.BlockSpec((d, h), lambda i: (0, 0)),
        ],
        out_specs=[
            pl.BlockSpec((b, 1), lambda i: (i, 0)),
            pl.BlockSpec((b, h), lambda i: (i, 0)),
        ],
        out_shape=[
            jax.ShapeDtypeStruct((n, 1), jnp.float32),
            jax.ShapeDtypeStruct((n, h), jnp.float32),
        ],
    )(d0, d1, x, W1)

    edge_k = _make_edge_kernel(npad, e, h, _K)

    p = edge_k(g1, src, dst, zero2)

    g2 = pl.pallas_call(
        _tc2_body,
        grid=grid,
        in_specs=[
            pl.BlockSpec((b, h), lambda i: (i, 0)),
            pl.BlockSpec((b, h), lambda i: (i, 0)),
            pl.BlockSpec((b, h), lambda i: (i, 0)),
            pl.BlockSpec((b, 1), lambda i: (i, 0)),
            pl.BlockSpec((1, h), lambda i: (0, 0)),
            pl.BlockSpec((h, h), lambda i: (0, 0)),
        ],
        out_specs=pl.BlockSpec((b, h), lambda i: (i, 0)),
        out_shape=jax.ShapeDtypeStruct((n, h), jnp.float32),
    )(p[:n], p[npad:npad + n], g1, dis, b1.reshape(1, h), W2)

    p2 = edge_k(g2, src, dst, zero2)

    out = pl.pallas_call(
        _tc3_body,
        grid=grid,
        in_specs=[
            pl.BlockSpec((b, h), lambda i: (i, 0)),
            pl.BlockSpec((b, h), lambda i: (i, 0)),
            pl.BlockSpec((b, h), lambda i: (i, 0)),
            pl.BlockSpec((b, 1), lambda i: (i, 0)),
            pl.BlockSpec((1, h), lambda i: (0, 0)),
            pl.BlockSpec((d, c), lambda i: (0, 0)),
            pl.BlockSpec((1, c), lambda i: (0, 0)),
        ],
        out_specs=pl.BlockSpec((b, c), lambda i: (i, 0)),
        out_shape=jax.ShapeDtypeStruct((n, c), jnp.float32),
    )(p2[:n], p2[npad:npad + n], g2, dis, b2.reshape(1, h), Wf, bf.reshape(1, c))

    return out


# speedup vs baseline: 29.5482x; 1.0744x over previous
import functools

import jax
import jax.numpy as jnp
from jax import lax
from jax.experimental import pallas as pl
from jax.experimental.pallas import tpu as pltpu
from jax.experimental.pallas import tpu_sc as plsc

_NC = 2
_NS = 16
_K = 128


@functools.lru_cache(maxsize=None)
def _make_deg_kernel(npad: int, e: int, k: int):
    ept = e // (_NC * _NS)
    nbatch = ept // k
    rpt = npad // _NS
    mesh = plsc.VectorSubcoreMesh(core_axis_name="c", subcore_axis_name="s")

    @functools.partial(
        pl.kernel,
        mesh=mesh,
        out_type=jax.ShapeDtypeStruct((_NC * npad,), jnp.float32),
        scratch_types=[
            pltpu.VMEM((_NB, k), jnp.int32),
            pltpu.VMEM((k,), jnp.float32),
            pltpu.VMEM_SHARED((npad,), jnp.float32),
        ] + [pltpu.SemaphoreType.DMA] * _NB,
    )
    def deg_k(dst_hbm, ones_hbm, zero_hbm, out_hbm, dstv, onesv, degs, *isem):
        cid = lax.axis_index("c")
        sid = lax.axis_index("s")
        r0 = sid * rpt
        pltpu.sync_copy(zero_hbm, degs.at[pl.ds(r0, rpt)])
        pltpu.sync_copy(ones_hbm, onesv)
        plsc.subcore_barrier()
        ebase = cid * (e // _NC) + sid * ept

        def start_idx(b, slot):
            off = pl.multiple_of(ebase + b * k, 8)
            pltpu.async_copy(dst_hbm.at[pl.ds(off, k)], dstv.at[slot],
                             isem[slot])

        def wait_idx(slot):
            pltpu.make_async_copy(dst_hbm.at[pl.ds(ebase, k)],
                                  dstv.at[slot], isem[slot]).wait()

        for b in range(_NB - 1):
            start_idx(b, b)

        def round_body(i, carry):
            b0 = i * _NB
            for s in range(_NB):
                start_idx(b0 + s + _NB - 1, (s + _NB - 1) % _NB)
                wait_idx(s)
                pltpu.sync_copy(onesv, degs.at[dstv.at[s]], add=True)
            return carry

        nround = (nbatch - (_NB - 1)) // _NB
        lax.fori_loop(0, nround, round_body, 0)
        for b in range(nround * _NB, nbatch):
            s = b % _NB
            if b + _NB - 1 < nbatch:
                start_idx(b + _NB - 1, (s + _NB - 1) % _NB)
            wait_idx(s)
            pltpu.sync_copy(onesv, degs.at[dstv.at[s]], add=True)

        plsc.subcore_barrier()
        pltpu.sync_copy(
            degs.at[pl.ds(r0, rpt)],
            out_hbm.at[pl.ds(cid * npad + r0, rpt)],
        )

    return deg_k


_NB = 3


@functools.lru_cache(maxsize=None)
def _make_edge_kernel(npad: int, e: int, h: int, k: int):
    ept = e // (_NC * _NS)
    rpt = npad // _NS
    nbatch = ept // k
    nround = nbatch // _NB - 1
    mesh = plsc.VectorSubcoreMesh(core_axis_name="c", subcore_axis_name="s")

    @functools.partial(
        pl.kernel,
        mesh=mesh,
        out_type=jax.ShapeDtypeStruct((_NC * npad, h), jnp.float32),
        scratch_types=[
            pltpu.VMEM((_NB, k), jnp.int32),
            pltpu.VMEM((_NB, k), jnp.int32),
            pltpu.VMEM((_NB, k, h), jnp.float32),
            pltpu.VMEM_SHARED((npad, h), jnp.float32),
        ] + [pltpu.SemaphoreType.DMA] * (2 * _NB),
    )
    def edge_k(g_hbm, src_hbm, dst_hbm, zero_hbm, out_hbm,
               srcv, dstv, rows, aggs, *sems):
        gsem = sems[:_NB]
        isem = sems[_NB:]
        cid = lax.axis_index("c")
        sid = lax.axis_index("s")
        r0 = sid * rpt
        pltpu.sync_copy(zero_hbm, aggs.at[pl.ds(r0, rpt)])
        plsc.subcore_barrier()
        ebase = cid * (e // _NC) + sid * ept

        def start_idx(b, slot):
            off = pl.multiple_of(ebase + b * k, 8)
            pltpu.async_copy(src_hbm.at[pl.ds(off, k)], srcv.at[slot],
                             isem[slot])
            pltpu.async_copy(dst_hbm.at[pl.ds(off, k)], dstv.at[slot],
                             isem[slot])

        def wait_idx(slot):
            pltpu.make_async_copy(src_hbm.at[pl.ds(ebase, k)],
                                  srcv.at[slot], isem[slot]).wait()
            pltpu.make_async_copy(dst_hbm.at[pl.ds(ebase, k)],
                                  dstv.at[slot], isem[slot]).wait()

        def start_gather(slot):
            pltpu.async_copy(g_hbm.at[srcv.at[slot]], rows.at[slot],
                             gsem[slot])

        def wait_gather(slot):
            pltpu.make_async_copy(g_hbm.at[srcv.at[slot]], rows.at[slot],
                                  gsem[slot]).wait()

        def scatter(slot):
            pltpu.sync_copy(rows.at[slot], aggs.at[dstv.at[slot]],
                            add=True)

        lk = _NB - 1

        for b in range(lk):
            start_idx(b, b)
        wait_idx(0)
        start_gather(0)

        def step(b, s):
            wait_idx((s + 1) % _NB)
            start_gather((s + 1) % _NB)
            wait_gather(s)
            scatter(s)

        def round_body(i, carry):
            b0 = i * _NB
            for s in range(_NB):
                start_idx(b0 + s + lk, (s + lk) % _NB)
                step(b0 + s, s)
            return carry

        lax.fori_loop(0, nround, round_body, 0)

        for b in range(nround * _NB, nbatch):
            s = b % _NB
            if b + lk < nbatch:
                start_idx(b + lk, (s + lk) % _NB)
            if b + 1 < nbatch:
                wait_idx((s + 1) % _NB)
                start_gather((s + 1) % _NB)
            wait_gather(s)
            scatter(s)

        plsc.subcore_barrier()
        pltpu.sync_copy(
            aggs.at[pl.ds(r0, rpt)],
            out_hbm.at[pl.ds(cid * npad + r0, rpt)],
        )

    return edge_k


def _tc1a_body(x_ref, w_ref, h_ref):
    h_ref[...] = jnp.dot(x_ref[...], w_ref[...],
                         preferred_element_type=jnp.float32)


def _tc1b_body(d0_ref, d1_ref, h_ref, dis_ref, g_ref):
    deg = d0_ref[...] + d1_ref[...] + 1.0
    dis = lax.rsqrt(deg)
    dis_ref[...] = dis
    g_ref[...] = h_ref[...] * dis


def _tc2_body(p0_ref, p1_ref, g_ref, dis_ref, b_ref, w_ref, out_ref):
    agg = p0_ref[...] + p1_ref[...] + g_ref[...]
    z = jnp.maximum(agg * dis_ref[...] + b_ref[...], 0.0)
    out_ref[...] = (
        jnp.dot(z, w_ref[...], preferred_element_type=jnp.float32)
        * dis_ref[...]
    )


def _tc3_body(p0_ref, p1_ref, g_ref, dis_ref, b_ref, wf_ref, bf_ref, out_ref):
    agg = p0_ref[...] + p1_ref[...] + g_ref[...]
    z = jnp.maximum(agg * dis_ref[...] + b_ref[...], 0.0)
    logits = (
        jnp.dot(z, wf_ref[...], preferred_element_type=jnp.float32)
        + bf_ref[...]
    )
    m = jnp.max(logits, axis=1, keepdims=True)
    s = logits - m
    out_ref[...] = s - jnp.log(jnp.sum(jnp.exp(s), axis=1, keepdims=True))


def _row_block(n):
    for b in (1000, 500, 250, 200, 128, 100, 50, 40, 8):
        if n % b == 0 and b % 8 == 0:
            return b
    return n


def kernel(x, edge_index, W1, b1, W2, b2, Wf, bf):
    n, d = x.shape
    h = W1.shape[1]
    c = Wf.shape[1]
    e = edge_index.shape[1]
    src = edge_index[0]
    dst = edge_index[1]

    npad = -(-n // 2048) * 2048
    napad = -(-n // 128) * 128
    if napad == n:
        napad = n + 128
    ep = -(-e // (_NC * _NS * _K)) * (_NC * _NS * _K)
    if ep > e:
        pad = ep - e
        pad_src = (jnp.arange(pad, dtype=jnp.int32)) % n
        pad_dst = n + (jnp.arange(pad, dtype=jnp.int32)) % (napad - n)
        src = jnp.concatenate([src, pad_src])
        dst = jnp.concatenate([dst, pad_dst])

    b = _row_block(n)
    grid = (n // b,)

    ones_k = jnp.ones((_K,), jnp.float32)
    zero1 = jnp.zeros((npad // _NS,), jnp.float32)
    zero2 = jnp.zeros((napad // _NS, h), jnp.float32)

    degf = _make_deg_kernel(npad, ep, _K)(dst, ones_k, zero1)
    d0 = degf[:n].reshape(n, 1)
    d1 = degf[npad:npad + n].reshape(n, 1)

    h1 = pl.pallas_call(
        _tc1a_body,
        grid=grid,
        in_specs=[
            pl.BlockSpec((b, d), lambda i: (i, 0)),
            pl.BlockSpec((d, h), lambda i: (0, 0)),
        ],
        out_specs=pl.BlockSpec((b, h), lambda i: (i, 0)),
        out_shape=jax.ShapeDtypeStruct((n, h), jnp.float32),
    )(x, W1)

    dis, g1 = pl.pallas_call(
        _tc1b_body,
        grid=grid,
        in_specs=[
            pl.BlockSpec((b, 1), lambda i: (i, 0)),
            pl.BlockSpec((b, 1), lambda i: (i, 0)),
            pl.BlockSpec((b, h), lambda i: (i, 0)),
        ],
        out_specs=[
            pl.BlockSpec((b, 1), lambda i: (i, 0)),
            pl.BlockSpec((b, h), lambda i: (i, 0)),
        ],
        out_shape=[
            jax.ShapeDtypeStruct((n, 1), jnp.float32),
            jax.ShapeDtypeStruct((n, h), jnp.float32),
        ],
    )(d0, d1, h1)

    edge_k = _make_edge_kernel(napad, ep, h, _K)

    p = edge_k(g1, src, dst, zero2)

    g2 = pl.pallas_call(
        _tc2_body,
        grid=grid,
        in_specs=[
            pl.BlockSpec((b, h), lambda i: (i, 0)),
            pl.BlockSpec((b, h), lambda i: (i, 0)),
            pl.BlockSpec((b, h), lambda i: (i, 0)),
            pl.BlockSpec((b, 1), lambda i: (i, 0)),
            pl.BlockSpec((1, h), lambda i: (0, 0)),
            pl.BlockSpec((h, h), lambda i: (0, 0)),
        ],
        out_specs=pl.BlockSpec((b, h), lambda i: (i, 0)),
        out_shape=jax.ShapeDtypeStruct((n, h), jnp.float32),
    )(p[:n], p[napad:napad + n], g1, dis, b1.reshape(1, h), W2)

    p2 = edge_k(g2, src, dst, zero2)

    out = pl.pallas_call(
        _tc3_body,
        grid=grid,
        in_specs=[
            pl.BlockSpec((b, h), lambda i: (i, 0)),
            pl.BlockSpec((b, h), lambda i: (i, 0)),
            pl.BlockSpec((b, h), lambda i: (i, 0)),
            pl.BlockSpec((b, 1), lambda i: (i, 0)),
            pl.BlockSpec((1, h), lambda i: (0, 0)),
            pl.BlockSpec((d, c), lambda i: (0, 0)),
            pl.BlockSpec((1, c), lambda i: (0, 0)),
        ],
        out_specs=pl.BlockSpec((b, c), lambda i: (i, 0)),
        out_shape=jax.ShapeDtypeStruct((n, c), jnp.float32),
    )(p2[:n], p2[napad:napad + n], g2, dis, b2.reshape(1, h), Wf, bf.reshape(1, c))

    return out


# speedup vs baseline: 30.5823x; 1.0350x over previous
import functools

import jax
import jax.numpy as jnp
from jax import lax
from jax.experimental import pallas as pl
from jax.experimental.pallas import tpu as pltpu
from jax.experimental.pallas import tpu_sc as plsc

_NC = 2
_NS = 16
_K = 80


@functools.lru_cache(maxsize=None)
def _make_deg_kernel(npad: int, e: int, k: int):
    ept = e // (_NC * _NS)
    nbatch = ept // k
    rpt = npad // _NS
    mesh = plsc.VectorSubcoreMesh(core_axis_name="c", subcore_axis_name="s")

    @functools.partial(
        pl.kernel,
        mesh=mesh,
        out_type=jax.ShapeDtypeStruct((_NC * npad,), jnp.float32),
        scratch_types=[
            pltpu.VMEM((_NB, k), jnp.int32),
            pltpu.VMEM((k,), jnp.float32),
            pltpu.VMEM_SHARED((npad,), jnp.float32),
        ] + [pltpu.SemaphoreType.DMA] * _NB,
    )
    def deg_k(dst_hbm, ones_hbm, zero_hbm, out_hbm, dstv, onesv, degs, *isem):
        cid = lax.axis_index("c")
        sid = lax.axis_index("s")
        r0 = sid * rpt
        pltpu.sync_copy(zero_hbm, degs.at[pl.ds(r0, rpt)])
        pltpu.sync_copy(ones_hbm, onesv)
        plsc.subcore_barrier()
        ebase = cid * (e // _NC) + sid * ept

        def start_idx(b, slot):
            off = pl.multiple_of(ebase + b * k, 8)
            pltpu.async_copy(dst_hbm.at[pl.ds(off, k)], dstv.at[slot],
                             isem[slot])

        def wait_idx(slot):
            pltpu.make_async_copy(dst_hbm.at[pl.ds(ebase, k)],
                                  dstv.at[slot], isem[slot]).wait()

        for b in range(_NB - 1):
            start_idx(b, b)

        def round_body(i, carry):
            b0 = i * _NB
            for s in range(_NB):
                start_idx(b0 + s + _NB - 1, (s + _NB - 1) % _NB)
                wait_idx(s)
                pltpu.sync_copy(onesv, degs.at[dstv.at[s]], add=True)
            return carry

        nround = (nbatch - (_NB - 1)) // _NB
        lax.fori_loop(0, nround, round_body, 0)
        for b in range(nround * _NB, nbatch):
            s = b % _NB
            if b + _NB - 1 < nbatch:
                start_idx(b + _NB - 1, (s + _NB - 1) % _NB)
            wait_idx(s)
            pltpu.sync_copy(onesv, degs.at[dstv.at[s]], add=True)

        plsc.subcore_barrier()
        pltpu.sync_copy(
            degs.at[pl.ds(r0, rpt)],
            out_hbm.at[pl.ds(cid * npad + r0, rpt)],
        )

    return deg_k


_NB = 4


@functools.lru_cache(maxsize=None)
def _make_edge_kernel(npad: int, e: int, h: int, k: int):
    ept = e // (_NC * _NS)
    rpt = npad // _NS
    nbatch = ept // k
    mesh = plsc.VectorSubcoreMesh(core_axis_name="c", subcore_axis_name="s")

    @functools.partial(
        pl.kernel,
        mesh=mesh,
        out_type=jax.ShapeDtypeStruct((_NC * npad, h), jnp.float32),
        scratch_types=[
            pltpu.VMEM((_NB, k), jnp.int32),
            pltpu.VMEM((_NB, k), jnp.int32),
            pltpu.VMEM((_NB, k, h), jnp.float32),
            pltpu.VMEM_SHARED((npad, h), jnp.float32),
        ] + [pltpu.SemaphoreType.DMA] * (3 * _NB),
    )
    def edge_k(g_hbm, src_hbm, dst_hbm, zero_hbm, out_hbm,
               srcv, dstv, rows, aggs, *sems):
        gsem = sems[:_NB]
        isem = sems[_NB:2 * _NB]
        ssem = sems[2 * _NB:]
        cid = lax.axis_index("c")
        sid = lax.axis_index("s")
        r0 = sid * rpt
        pltpu.sync_copy(zero_hbm, aggs.at[pl.ds(r0, rpt)])
        plsc.subcore_barrier()
        ebase = cid * (e // _NC) + sid * ept

        def start_idx(b, slot):
            off = pl.multiple_of(ebase + b * k, 8)
            pltpu.async_copy(src_hbm.at[pl.ds(off, k)], srcv.at[slot],
                             isem[slot])
            pltpu.async_copy(dst_hbm.at[pl.ds(off, k)], dstv.at[slot],
                             isem[slot])

        def wait_idx(slot):
            pltpu.make_async_copy(src_hbm.at[pl.ds(ebase, k)],
                                  srcv.at[slot], isem[slot]).wait()
            pltpu.make_async_copy(dst_hbm.at[pl.ds(ebase, k)],
                                  dstv.at[slot], isem[slot]).wait()

        def start_gather(slot):
            pltpu.async_copy(g_hbm.at[srcv.at[slot]], rows.at[slot],
                             gsem[slot])

        def wait_gather(slot):
            pltpu.make_async_copy(g_hbm.at[srcv.at[slot]], rows.at[slot],
                                  gsem[slot]).wait()

        def start_scatter(slot):
            pltpu.async_copy(rows.at[slot], aggs.at[dstv.at[slot]],
                             ssem[slot], add=True)

        def wait_scatter(slot):
            pltpu.make_async_copy(rows.at[slot], aggs.at[dstv.at[slot]],
                                  ssem[slot]).wait()

        start_idx(0, 0)
        start_idx(1, 1)
        wait_idx(0)
        start_gather(0)
        for b in (0, 1):
            start_idx(b + 2, b + 2)
            wait_idx(b + 1)
            start_gather(b + 1)
            wait_gather(b)
            start_scatter(b)

        def step(b, s):
            wait_scatter((s + 2) % _NB)
            start_idx(b + 2, (s + 2) % _NB)
            wait_idx((s + 1) % _NB)
            start_gather((s + 1) % _NB)
            wait_gather(s)
            start_scatter(s)

        nround = (nbatch - 2 - 3) // _NB
        def round_body(i, carry):
            b0 = 2 + i * _NB
            for s in range(_NB):
                step(b0 + s, (2 + s) % _NB)
            return carry

        lax.fori_loop(0, nround, round_body, 0)

        for b in range(2 + nround * _NB, nbatch):
            s = b % _NB
            wait_scatter((s + 2) % _NB)
            if b + 2 < nbatch:
                start_idx(b + 2, (s + 2) % _NB)
            if b + 1 < nbatch:
                wait_idx((s + 1) % _NB)
                start_gather((s + 1) % _NB)
            wait_gather(s)
            start_scatter(s)
        wait_scatter((nbatch - 2) % _NB)
        wait_scatter((nbatch - 1) % _NB)

        plsc.subcore_barrier()
        pltpu.sync_copy(
            aggs.at[pl.ds(r0, rpt)],
            out_hbm.at[pl.ds(cid * npad + r0, rpt)],
        )

    return edge_k


def _tc1a_body(x_ref, w_ref, h_ref):
    h_ref[...] = jnp.dot(x_ref[...], w_ref[...],
                         preferred_element_type=jnp.float32)


def _tc1b_body(d0_ref, d1_ref, h_ref, dis_ref, g_ref):
    deg = d0_ref[...] + d1_ref[...] + 1.0
    dis = lax.rsqrt(deg)
    dis_ref[...] = dis
    g_ref[...] = h_ref[...] * dis


def _tc2_body(p0_ref, p1_ref, g_ref, dis_ref, b_ref, w_ref, out_ref):
    agg = p0_ref[...] + p1_ref[...] + g_ref[...]
    z = jnp.maximum(agg * dis_ref[...] + b_ref[...], 0.0)
    out_ref[...] = (
        jnp.dot(z, w_ref[...], preferred_element_type=jnp.float32)
        * dis_ref[...]
    )


def _tc3_body(p0_ref, p1_ref, g_ref, dis_ref, b_ref, wf_ref, bf_ref, out_ref):
    agg = p0_ref[...] + p1_ref[...] + g_ref[...]
    z = jnp.maximum(agg * dis_ref[...] + b_ref[...], 0.0)
    logits = (
        jnp.dot(z, wf_ref[...], preferred_element_type=jnp.float32)
        + bf_ref[...]
    )
    m = jnp.max(logits, axis=1, keepdims=True)
    s = logits - m
    out_ref[...] = s - jnp.log(jnp.sum(jnp.exp(s), axis=1, keepdims=True))


def _row_block(n):
    for b in (1000, 500, 250, 200, 128, 100, 50, 40, 8):
        if n % b == 0 and b % 8 == 0:
            return b
    return n


def kernel(x, edge_index, W1, b1, W2, b2, Wf, bf):
    n, d = x.shape
    h = W1.shape[1]
    c = Wf.shape[1]
    e = edge_index.shape[1]
    src = edge_index[0]
    dst = edge_index[1]

    npad = -(-n // 2048) * 2048
    napad = -(-n // 128) * 128
    if napad == n:
        napad = n + 128
    ep = -(-e // (_NC * _NS * _K)) * (_NC * _NS * _K)
    if ep > e:
        pad = ep - e
        pad_src = (jnp.arange(pad, dtype=jnp.int32)) % n
        pad_dst = n + (jnp.arange(pad, dtype=jnp.int32)) % (napad - n)
        src = jnp.concatenate([src, pad_src])
        dst = jnp.concatenate([dst, pad_dst])

    b = _row_block(n)
    grid = (n // b,)

    ones_k = jnp.ones((_K,), jnp.float32)
    zero1 = jnp.zeros((npad // _NS,), jnp.float32)
    zero2 = jnp.zeros((napad // _NS, h), jnp.float32)

    degf = _make_deg_kernel(npad, ep, _K)(dst, ones_k, zero1)
    d0 = degf[:n].reshape(n, 1)
    d1 = degf[npad:npad + n].reshape(n, 1)

    h1 = pl.pallas_call(
        _tc1a_body,
        grid=grid,
        in_specs=[
            pl.BlockSpec((b, d), lambda i: (i, 0)),
            pl.BlockSpec((d, h), lambda i: (0, 0)),
        ],
        out_specs=pl.BlockSpec((b, h), lambda i: (i, 0)),
        out_shape=jax.ShapeDtypeStruct((n, h), jnp.float32),
    )(x, W1)

    dis, g1 = pl.pallas_call(
        _tc1b_body,
        grid=grid,
        in_specs=[
            pl.BlockSpec((b, 1), lambda i: (i, 0)),
            pl.BlockSpec((b, 1), lambda i: (i, 0)),
            pl.BlockSpec((b, h), lambda i: (i, 0)),
        ],
        out_specs=[
            pl.BlockSpec((b, 1), lambda i: (i, 0)),
            pl.BlockSpec((b, h), lambda i: (i, 0)),
        ],
        out_shape=[
            jax.ShapeDtypeStruct((n, 1), jnp.float32),
            jax.ShapeDtypeStruct((n, h), jnp.float32),
        ],
    )(d0, d1, h1)

    edge_k = _make_edge_kernel(napad, ep, h, _K)

    p = edge_k(g1, src, dst, zero2)

    g2 = pl.pallas_call(
        _tc2_body,
        grid=grid,
        in_specs=[
            pl.BlockSpec((b, h), lambda i: (i, 0)),
            pl.BlockSpec((b, h), lambda i: (i, 0)),
            pl.BlockSpec((b, h), lambda i: (i, 0)),
            pl.BlockSpec((b, 1), lambda i: (i, 0)),
            pl.BlockSpec((1, h), lambda i: (0, 0)),
            pl.BlockSpec((h, h), lambda i: (0, 0)),
        ],
        out_specs=pl.BlockSpec((b, h), lambda i: (i, 0)),
        out_shape=jax.ShapeDtypeStruct((n, h), jnp.float32),
    )(p[:n], p[napad:napad + n], g1, dis, b1.reshape(1, h), W2)

    p2 = edge_k(g2, src, dst, zero2)

    out = pl.pallas_call(
        _tc3_body,
        grid=grid,
        in_specs=[
            pl.BlockSpec((b, h), lambda i: (i, 0)),
            pl.BlockSpec((b, h), lambda i: (i, 0)),
            pl.BlockSpec((b, h), lambda i: (i, 0)),
            pl.BlockSpec((b, 1), lambda i: (i, 0)),
            pl.BlockSpec((1, h), lambda i: (0, 0)),
            pl.BlockSpec((d, c), lambda i: (0, 0)),
            pl.BlockSpec((1, c), lambda i: (0, 0)),
        ],
        out_specs=pl.BlockSpec((b, c), lambda i: (i, 0)),
        out_shape=jax.ShapeDtypeStruct((n, c), jnp.float32),
    )(p2[:n], p2[napad:napad + n], g2, dis, b2.reshape(1, h), Wf, bf.reshape(1, c))

    return out


# speedup vs baseline: 31.1196x; 1.0176x over previous
import functools

import jax
import jax.numpy as jnp
from jax import lax
from jax.experimental import pallas as pl
from jax.experimental.pallas import tpu as pltpu
from jax.experimental.pallas import tpu_sc as plsc

_NC = 2
_NS = 16
_K = 80


@functools.lru_cache(maxsize=None)
def _make_deg_kernel(npad: int, e: int, k: int):
    ept = e // (_NC * _NS)
    nbatch = ept // k
    rpt = npad // _NS
    mesh = plsc.VectorSubcoreMesh(core_axis_name="c", subcore_axis_name="s")

    @functools.partial(
        pl.kernel,
        mesh=mesh,
        out_type=jax.ShapeDtypeStruct((_NC * npad,), jnp.float32),
        scratch_types=[
            pltpu.VMEM((_NB, k), jnp.int32),
            pltpu.VMEM((k,), jnp.float32),
            pltpu.VMEM_SHARED((npad,), jnp.float32),
        ] + [pltpu.SemaphoreType.DMA] * _NB,
    )
    def deg_k(dst_hbm, ones_hbm, zero_hbm, out_hbm, dstv, onesv, degs, *isem):
        cid = lax.axis_index("c")
        sid = lax.axis_index("s")
        r0 = sid * rpt
        pltpu.sync_copy(zero_hbm, degs.at[pl.ds(r0, rpt)])
        pltpu.sync_copy(ones_hbm, onesv)
        plsc.subcore_barrier()
        ebase = cid * (e // _NC) + sid * ept

        def start_idx(b, slot):
            off = pl.multiple_of(ebase + b * k, 8)
            pltpu.async_copy(dst_hbm.at[pl.ds(off, k)], dstv.at[slot],
                             isem[slot])

        def wait_idx(slot):
            pltpu.make_async_copy(dst_hbm.at[pl.ds(ebase, k)],
                                  dstv.at[slot], isem[slot]).wait()

        for b in range(_NB - 1):
            start_idx(b, b)

        def round_body(i, carry):
            b0 = i * _NB
            for s in range(_NB):
                start_idx(b0 + s + _NB - 1, (s + _NB - 1) % _NB)
                wait_idx(s)
                pltpu.sync_copy(onesv, degs.at[dstv.at[s]], add=True)
            return carry

        nround = (nbatch - (_NB - 1)) // _NB
        lax.fori_loop(0, nround, round_body, 0)
        for b in range(nround * _NB, nbatch):
            s = b % _NB
            if b + _NB - 1 < nbatch:
                start_idx(b + _NB - 1, (s + _NB - 1) % _NB)
            wait_idx(s)
            pltpu.sync_copy(onesv, degs.at[dstv.at[s]], add=True)

        plsc.subcore_barrier()
        pltpu.sync_copy(
            degs.at[pl.ds(r0, rpt)],
            out_hbm.at[pl.ds(cid * npad + r0, rpt)],
        )

    return deg_k


_NB = 4


@functools.lru_cache(maxsize=None)
def _make_edge_kernel(npad: int, e: int, h: int, k: int):
    ept = e // (_NC * _NS)
    rpt = npad // _NS
    nbatch = ept // k
    mesh = plsc.VectorSubcoreMesh(core_axis_name="c", subcore_axis_name="s")

    @functools.partial(
        pl.kernel,
        mesh=mesh,
        out_type=jax.ShapeDtypeStruct((_NC * npad, h), jnp.float32),
        scratch_types=[
            pltpu.VMEM((_NB, k), jnp.int32),
            pltpu.VMEM((_NB, k), jnp.int32),
            pltpu.VMEM((_NB, k, h), jnp.float32),
            pltpu.VMEM_SHARED((npad, h), jnp.float32),
        ] + [pltpu.SemaphoreType.DMA] * (3 * _NB),
    )
    def edge_k(g_hbm, src_hbm, dst_hbm, zero_hbm, out_hbm,
               srcv, dstv, rows, aggs, *sems):
        gsem = sems[:_NB]
        isem = sems[_NB:2 * _NB]
        ssem = sems[2 * _NB:]
        cid = lax.axis_index("c")
        sid = lax.axis_index("s")
        r0 = sid * rpt
        pltpu.sync_copy(zero_hbm, aggs.at[pl.ds(r0, rpt)])
        plsc.subcore_barrier()
        ebase = cid * (e // _NC) + sid * ept

        def start_idx(b, slot):
            off = pl.multiple_of(ebase + b * k, 8)
            pltpu.async_copy(src_hbm.at[pl.ds(off, k)], srcv.at[slot],
                             isem[slot])
            pltpu.async_copy(dst_hbm.at[pl.ds(off, k)], dstv.at[slot],
                             isem[slot])

        def wait_idx(slot):
            pltpu.make_async_copy(src_hbm.at[pl.ds(ebase, k)],
                                  srcv.at[slot], isem[slot]).wait()
            pltpu.make_async_copy(dst_hbm.at[pl.ds(ebase, k)],
                                  dstv.at[slot], isem[slot]).wait()

        def start_gather(slot):
            pltpu.async_copy(g_hbm.at[srcv.at[slot]], rows.at[slot],
                             gsem[slot])

        def wait_gather(slot):
            pltpu.make_async_copy(g_hbm.at[srcv.at[slot]], rows.at[slot],
                                  gsem[slot]).wait()

        def start_scatter(slot):
            pltpu.async_copy(rows.at[slot], aggs.at[dstv.at[slot]],
                             ssem[slot], add=True)

        def wait_scatter(slot):
            pltpu.make_async_copy(rows.at[slot], aggs.at[dstv.at[slot]],
                                  ssem[slot]).wait()

        start_idx(0, 0)
        start_idx(1, 1)
        wait_idx(0)
        start_gather(0)
        for b in (0, 1):
            start_idx(b + 2, b + 2)
            wait_idx(b + 1)
            start_gather(b + 1)
            wait_gather(b)
            start_scatter(b)

        def step(b, s):
            wait_scatter((s + 2) % _NB)
            start_idx(b + 2, (s + 2) % _NB)
            wait_idx((s + 1) % _NB)
            start_gather((s + 1) % _NB)
            wait_gather(s)
            start_scatter(s)

        nround = (nbatch - 2 - 3) // _NB
        def round_body(i, carry):
            b0 = 2 + i * _NB
            for s in range(_NB):
                step(b0 + s, (2 + s) % _NB)
            return carry

        lax.fori_loop(0, nround, round_body, 0)

        for b in range(2 + nround * _NB, nbatch):
            s = b % _NB
            wait_scatter((s + 2) % _NB)
            if b + 2 < nbatch:
                start_idx(b + 2, (s + 2) % _NB)
            if b + 1 < nbatch:
                wait_idx((s + 1) % _NB)
                start_gather((s + 1) % _NB)
            wait_gather(s)
            start_scatter(s)
        wait_scatter((nbatch - 2) % _NB)
        wait_scatter((nbatch - 1) % _NB)

        plsc.subcore_barrier()
        pltpu.sync_copy(
            aggs.at[pl.ds(r0, rpt)],
            out_hbm.at[pl.ds(cid * npad + r0, rpt)],
        )

    return edge_k


def _tc1a_body(x_ref, w_ref, h_ref):
    h_ref[...] = jnp.dot(x_ref[...], w_ref[...],
                         preferred_element_type=jnp.float32)


def _tc1b_body(d0_ref, d1_ref, h_ref, dis_ref, g_ref):
    deg = d0_ref[...] + d1_ref[...] + 1.0
    dis = lax.rsqrt(deg)
    dis_ref[...] = dis
    g_ref[...] = h_ref[...] * dis


def _tc2_body(p0_ref, p1_ref, g_ref, dis_ref, b_ref, w_ref, out_ref):
    agg = p0_ref[...] + p1_ref[...] + g_ref[...]
    z = jnp.maximum(agg * dis_ref[...] + b_ref[...], 0.0)
    out_ref[...] = (
        jnp.dot(z, w_ref[...], preferred_element_type=jnp.float32)
        * dis_ref[...]
    )


def _tc3_body(p0_ref, p1_ref, g_ref, dis_ref, b_ref, wf_ref, bf_ref, out_ref):
    agg = p0_ref[...] + p1_ref[...] + g_ref[...]
    z = jnp.maximum(agg * dis_ref[...] + b_ref[...], 0.0)
    logits = (
        jnp.dot(z, wf_ref[...], preferred_element_type=jnp.float32)
        + bf_ref[...]
    )
    m = jnp.max(logits, axis=1, keepdims=True)
    s = logits - m
    out_ref[...] = s - jnp.log(jnp.sum(jnp.exp(s), axis=1, keepdims=True))


def _row_block(n):
    for b in (2000, 1000, 500, 250, 200, 128, 100, 50, 40, 8):
        if n % b == 0 and b % 8 == 0:
            return b
    return n


def kernel(x, edge_index, W1, b1, W2, b2, Wf, bf):
    n, d = x.shape
    h = W1.shape[1]
    c = Wf.shape[1]
    e = edge_index.shape[1]
    src = edge_index[0]
    dst = edge_index[1]

    npad = -(-n // 2048) * 2048
    napad = -(-n // 128) * 128
    if napad == n:
        napad = n + 128
    ep = -(-e // (_NC * _NS * _K)) * (_NC * _NS * _K)
    if ep > e:
        pad = ep - e
        pad_src = (jnp.arange(pad, dtype=jnp.int32)) % n
        pad_dst = n + (jnp.arange(pad, dtype=jnp.int32)) % (napad - n)
        src = jnp.concatenate([src, pad_src])
        dst = jnp.concatenate([dst, pad_dst])

    b = _row_block(n)
    grid = (n // b,)

    ones_k = jnp.ones((_K,), jnp.float32)
    zero1 = jnp.zeros((npad // _NS,), jnp.float32)
    zero2 = jnp.zeros((napad // _NS, h), jnp.float32)

    degf = _make_deg_kernel(npad, ep, _K)(dst, ones_k, zero1)
    d0 = degf[:n].reshape(n, 1)
    d1 = degf[npad:npad + n].reshape(n, 1)

    h1 = pl.pallas_call(
        _tc1a_body,
        grid=grid,
        in_specs=[
            pl.BlockSpec((b, d), lambda i: (i, 0)),
            pl.BlockSpec((d, h), lambda i: (0, 0)),
        ],
        out_specs=pl.BlockSpec((b, h), lambda i: (i, 0)),
        out_shape=jax.ShapeDtypeStruct((n, h), jnp.float32),
    )(x, W1)

    dis, g1 = pl.pallas_call(
        _tc1b_body,
        grid=grid,
        in_specs=[
            pl.BlockSpec((b, 1), lambda i: (i, 0)),
            pl.BlockSpec((b, 1), lambda i: (i, 0)),
            pl.BlockSpec((b, h), lambda i: (i, 0)),
        ],
        out_specs=[
            pl.BlockSpec((b, 1), lambda i: (i, 0)),
            pl.BlockSpec((b, h), lambda i: (i, 0)),
        ],
        out_shape=[
            jax.ShapeDtypeStruct((n, 1), jnp.float32),
            jax.ShapeDtypeStruct((n, h), jnp.float32),
        ],
    )(d0, d1, h1)

    edge_k = _make_edge_kernel(napad, ep, h, _K)

    p = edge_k(g1, src, dst, zero2)

    g2 = pl.pallas_call(
        _tc2_body,
        grid=grid,
        in_specs=[
            pl.BlockSpec((b, h), lambda i: (i, 0)),
            pl.BlockSpec((b, h), lambda i: (i, 0)),
            pl.BlockSpec((b, h), lambda i: (i, 0)),
            pl.BlockSpec((b, 1), lambda i: (i, 0)),
            pl.BlockSpec((1, h), lambda i: (0, 0)),
            pl.BlockSpec((h, h), lambda i: (0, 0)),
        ],
        out_specs=pl.BlockSpec((b, h), lambda i: (i, 0)),
        out_shape=jax.ShapeDtypeStruct((n, h), jnp.float32),
    )(p[:n], p[napad:napad + n], g1, dis, b1.reshape(1, h), W2)

    p2 = edge_k(g2, src, dst, zero2)

    out = pl.pallas_call(
        _tc3_body,
        grid=grid,
        in_specs=[
            pl.BlockSpec((b, h), lambda i: (i, 0)),
            pl.BlockSpec((b, h), lambda i: (i, 0)),
            pl.BlockSpec((b, h), lambda i: (i, 0)),
            pl.BlockSpec((b, 1), lambda i: (i, 0)),
            pl.BlockSpec((1, h), lambda i: (0, 0)),
            pl.BlockSpec((d, c), lambda i: (0, 0)),
            pl.BlockSpec((1, c), lambda i: (0, 0)),
        ],
        out_specs=pl.BlockSpec((b, c), lambda i: (i, 0)),
        out_shape=jax.ShapeDtypeStruct((n, c), jnp.float32),
    )(p2[:n], p2[napad:napad + n], g2, dis, b2.reshape(1, h), Wf, bf.reshape(1, c))

    return out


# speedup vs baseline: 31.5312x; 1.0132x over previous
import functools

import jax
import jax.numpy as jnp
from jax import lax
from jax.experimental import pallas as pl
from jax.experimental.pallas import tpu as pltpu
from jax.experimental.pallas import tpu_sc as plsc

_NC = 2
_NS = 16
_K = 80


@functools.lru_cache(maxsize=None)
def _make_deg_kernel(npad: int, e: int, k: int):
    ept = e // (_NC * _NS)
    nbatch = ept // k
    rpt = npad // _NS
    mesh = plsc.VectorSubcoreMesh(core_axis_name="c", subcore_axis_name="s")

    @functools.partial(
        pl.kernel,
        mesh=mesh,
        out_type=jax.ShapeDtypeStruct((_NC * npad,), jnp.float32),
        scratch_types=[
            pltpu.VMEM((_NB, k), jnp.int32),
            pltpu.VMEM((k,), jnp.float32),
            pltpu.VMEM_SHARED((npad,), jnp.float32),
        ] + [pltpu.SemaphoreType.DMA] * _NB,
    )
    def deg_k(dst_hbm, ones_hbm, zero_hbm, out_hbm, dstv, onesv, degs, *isem):
        cid = lax.axis_index("c")
        sid = lax.axis_index("s")
        r0 = sid * rpt
        pltpu.sync_copy(zero_hbm, degs.at[pl.ds(r0, rpt)])
        pltpu.sync_copy(ones_hbm, onesv)
        plsc.subcore_barrier()
        ebase = cid * (e // _NC) + sid * ept

        def start_idx(b, slot):
            off = pl.multiple_of(ebase + b * k, 8)
            pltpu.async_copy(dst_hbm.at[pl.ds(off, k)], dstv.at[slot],
                             isem[slot])

        def wait_idx(slot):
            pltpu.make_async_copy(dst_hbm.at[pl.ds(ebase, k)],
                                  dstv.at[slot], isem[slot]).wait()

        for b in range(_NB - 1):
            start_idx(b, b)

        def round_body(i, carry):
            b0 = i * _NB
            for s in range(_NB):
                start_idx(b0 + s + _NB - 1, (s + _NB - 1) % _NB)
                wait_idx(s)
                pltpu.sync_copy(onesv, degs.at[dstv.at[s]], add=True)
            return carry

        nround = (nbatch - (_NB - 1)) // _NB
        lax.fori_loop(0, nround, round_body, 0)
        for b in range(nround * _NB, nbatch):
            s = b % _NB
            if b + _NB - 1 < nbatch:
                start_idx(b + _NB - 1, (s + _NB - 1) % _NB)
            wait_idx(s)
            pltpu.sync_copy(onesv, degs.at[dstv.at[s]], add=True)

        plsc.subcore_barrier()
        pltpu.sync_copy(
            degs.at[pl.ds(r0, rpt)],
            out_hbm.at[pl.ds(cid * npad + r0, rpt)],
        )

    return deg_k


_NB = 4


@functools.lru_cache(maxsize=None)
def _make_edge_kernel(npad: int, e: int, h: int, k: int):
    ept = e // (_NC * _NS)
    rpt = npad // _NS
    nbatch = ept // k
    mesh = plsc.VectorSubcoreMesh(core_axis_name="c", subcore_axis_name="s")

    @functools.partial(
        pl.kernel,
        mesh=mesh,
        out_type=jax.ShapeDtypeStruct((_NC * npad, h), jnp.float32),
        scratch_types=[
            pltpu.VMEM((_NB, k), jnp.int32),
            pltpu.VMEM((_NB, k), jnp.int32),
            pltpu.VMEM((_NB, k, h), jnp.float32),
            pltpu.VMEM_SHARED((npad, h), jnp.float32),
        ] + [pltpu.SemaphoreType.DMA] * (3 * _NB),
    )
    def edge_k(g_hbm, src_hbm, dst_hbm, zero_hbm, out_hbm,
               srcv, dstv, rows, aggs, *sems):
        gsem = sems[:_NB]
        isem = sems[_NB:2 * _NB]
        ssem = sems[2 * _NB:]
        cid = lax.axis_index("c")
        sid = lax.axis_index("s")
        r0 = sid * rpt
        pltpu.sync_copy(zero_hbm, aggs.at[pl.ds(r0, rpt)])
        plsc.subcore_barrier()
        ebase = cid * (e // _NC) + sid * ept

        def start_idx(b, slot):
            off = pl.multiple_of(ebase + b * k, 8)
            pltpu.async_copy(src_hbm.at[pl.ds(off, k)], srcv.at[slot],
                             isem[slot])
            pltpu.async_copy(dst_hbm.at[pl.ds(off, k)], dstv.at[slot],
                             isem[slot])

        def wait_idx(slot):
            pltpu.make_async_copy(src_hbm.at[pl.ds(ebase, k)],
                                  srcv.at[slot], isem[slot]).wait()
            pltpu.make_async_copy(dst_hbm.at[pl.ds(ebase, k)],
                                  dstv.at[slot], isem[slot]).wait()

        def start_gather(slot):
            pltpu.async_copy(g_hbm.at[srcv.at[slot]], rows.at[slot],
                             gsem[slot])

        def wait_gather(slot):
            pltpu.make_async_copy(g_hbm.at[srcv.at[slot]], rows.at[slot],
                                  gsem[slot]).wait()

        def start_scatter(slot):
            pltpu.async_copy(rows.at[slot], aggs.at[dstv.at[slot]],
                             ssem[slot], add=True)

        def wait_scatter(slot):
            pltpu.make_async_copy(rows.at[slot], aggs.at[dstv.at[slot]],
                                  ssem[slot]).wait()

        start_idx(0, 0)
        start_idx(1, 1)
        wait_idx(0)
        start_gather(0)
        for b in (0, 1):
            start_idx(b + 2, b + 2)
            wait_idx(b + 1)
            start_gather(b + 1)
            wait_gather(b)
            start_scatter(b)

        def step(b, s):
            wait_scatter((s + 2) % _NB)
            start_idx(b + 2, (s + 2) % _NB)
            wait_idx((s + 1) % _NB)
            start_gather((s + 1) % _NB)
            wait_gather(s)
            start_scatter(s)

        nround = (nbatch - 2 - 3) // _NB
        def round_body(i, carry):
            b0 = 2 + i * _NB
            for s in range(_NB):
                step(b0 + s, (2 + s) % _NB)
            return carry

        lax.fori_loop(0, nround, round_body, 0)

        for b in range(2 + nround * _NB, nbatch):
            s = b % _NB
            wait_scatter((s + 2) % _NB)
            if b + 2 < nbatch:
                start_idx(b + 2, (s + 2) % _NB)
            if b + 1 < nbatch:
                wait_idx((s + 1) % _NB)
                start_gather((s + 1) % _NB)
            wait_gather(s)
            start_scatter(s)
        wait_scatter((nbatch - 2) % _NB)
        wait_scatter((nbatch - 1) % _NB)

        plsc.subcore_barrier()
        pltpu.sync_copy(
            aggs.at[pl.ds(r0, rpt)],
            out_hbm.at[pl.ds(cid * npad + r0, rpt)],
        )

    return edge_k


def _tc1a_body(x_ref, w_ref, h_ref):
    h_ref[...] = jnp.dot(x_ref[...], w_ref[...],
                         preferred_element_type=jnp.float32)


def _tc1b_body(d0_ref, d1_ref, h_ref, dis_ref, g_ref):
    deg = d0_ref[...] + d1_ref[...] + 1.0
    dis = lax.rsqrt(deg)
    dis_ref[...] = dis
    g_ref[...] = h_ref[...] * dis


def _tc2_body(p0_ref, p1_ref, g_ref, dis_ref, b_ref, w_ref, out_ref):
    agg = p0_ref[...] + p1_ref[...] + g_ref[...]
    z = jnp.maximum(agg * dis_ref[...] + b_ref[...], 0.0)
    out_ref[...] = (
        jnp.dot(z, w_ref[...], preferred_element_type=jnp.float32)
        * dis_ref[...]
    )


def _tc3_body(p0_ref, p1_ref, g_ref, dis_ref, b_ref, wf_ref, bf_ref, out_ref):
    agg = p0_ref[...] + p1_ref[...] + g_ref[...]
    z = jnp.maximum(agg * dis_ref[...] + b_ref[...], 0.0)
    logits = (
        jnp.dot(z, wf_ref[...], preferred_element_type=jnp.float32)
        + bf_ref[...]
    )
    m = jnp.max(logits, axis=1, keepdims=True)
    s = logits - m
    out_ref[...] = s - jnp.log(jnp.sum(jnp.exp(s), axis=1, keepdims=True))


def _row_block(n):
    for b in (5000, 2000, 1000, 500, 250, 200, 128, 100, 50, 40, 8):
        if n % b == 0 and b % 8 == 0:
            return b
    return n


def kernel(x, edge_index, W1, b1, W2, b2, Wf, bf):
    n, d = x.shape
    h = W1.shape[1]
    c = Wf.shape[1]
    e = edge_index.shape[1]
    src = edge_index[0]
    dst = edge_index[1]

    npad = -(-n // 2048) * 2048
    napad = -(-n // 128) * 128
    if napad == n:
        napad = n + 128
    ep = -(-e // (_NC * _NS * _K)) * (_NC * _NS * _K)
    if ep > e:
        pad = ep - e
        pad_src = (jnp.arange(pad, dtype=jnp.int32)) % n
        pad_dst = n + (jnp.arange(pad, dtype=jnp.int32)) % (napad - n)
        src = jnp.concatenate([src, pad_src])
        dst = jnp.concatenate([dst, pad_dst])

    b = _row_block(n)
    grid = (n // b,)

    ones_k = jnp.ones((_K,), jnp.float32)
    zero1 = jnp.zeros((npad // _NS,), jnp.float32)
    zero2 = jnp.zeros((napad // _NS, h), jnp.float32)

    degf = _make_deg_kernel(npad, ep, _K)(dst, ones_k, zero1)
    d0 = degf[:n].reshape(n, 1)
    d1 = degf[npad:npad + n].reshape(n, 1)

    h1 = pl.pallas_call(
        _tc1a_body,
        grid=grid,
        in_specs=[
            pl.BlockSpec((b, d), lambda i: (i, 0)),
            pl.BlockSpec((d, h), lambda i: (0, 0)),
        ],
        out_specs=pl.BlockSpec((b, h), lambda i: (i, 0)),
        out_shape=jax.ShapeDtypeStruct((n, h), jnp.float32),
    )(x, W1)

    dis, g1 = pl.pallas_call(
        _tc1b_body,
        grid=grid,
        in_specs=[
            pl.BlockSpec((b, 1), lambda i: (i, 0)),
            pl.BlockSpec((b, 1), lambda i: (i, 0)),
            pl.BlockSpec((b, h), lambda i: (i, 0)),
        ],
        out_specs=[
            pl.BlockSpec((b, 1), lambda i: (i, 0)),
            pl.BlockSpec((b, h), lambda i: (i, 0)),
        ],
        out_shape=[
            jax.ShapeDtypeStruct((n, 1), jnp.float32),
            jax.ShapeDtypeStruct((n, h), jnp.float32),
        ],
    )(d0, d1, h1)

    edge_k = _make_edge_kernel(napad, ep, h, _K)

    p = edge_k(g1, src, dst, zero2)

    g2 = pl.pallas_call(
        _tc2_body,
        grid=grid,
        in_specs=[
            pl.BlockSpec((b, h), lambda i: (i, 0)),
            pl.BlockSpec((b, h), lambda i: (i, 0)),
            pl.BlockSpec((b, h), lambda i: (i, 0)),
            pl.BlockSpec((b, 1), lambda i: (i, 0)),
            pl.BlockSpec((1, h), lambda i: (0, 0)),
            pl.BlockSpec((h, h), lambda i: (0, 0)),
        ],
        out_specs=pl.BlockSpec((b, h), lambda i: (i, 0)),
        out_shape=jax.ShapeDtypeStruct((n, h), jnp.float32),
    )(p[:n], p[napad:napad + n], g1, dis, b1.reshape(1, h), W2)

    p2 = edge_k(g2, src, dst, zero2)

    out = pl.pallas_call(
        _tc3_body,
        grid=grid,
        in_specs=[
            pl.BlockSpec((b, h), lambda i: (i, 0)),
            pl.BlockSpec((b, h), lambda i: (i, 0)),
            pl.BlockSpec((b, h), lambda i: (i, 0)),
            pl.BlockSpec((b, 1), lambda i: (i, 0)),
            pl.BlockSpec((1, h), lambda i: (0, 0)),
            pl.BlockSpec((d, c), lambda i: (0, 0)),
            pl.BlockSpec((1, c), lambda i: (0, 0)),
        ],
        out_specs=pl.BlockSpec((b, c), lambda i: (i, 0)),
        out_shape=jax.ShapeDtypeStruct((n, c), jnp.float32),
    )(p2[:n], p2[napad:napad + n], g2, dis, b2.reshape(1, h), Wf, bf.reshape(1, c))

    return out
